# baseline (speedup 1.0000x reference)
import jax
import jax.numpy as jnp
import numpy as np
from jax.experimental import pallas as pl
from jax.experimental.pallas import tpu as pltpu

S, H = 2048, 1024
NH, NKV, HD = 16, 8, 64
I, E = 2048, 8
EPS = 1e-6
EP = 16


def _rms_mm_kernel(x_ref, ln_ref, w_ref, o_ref):
    x = x_ref[...]
    h = x * jax.lax.rsqrt(jnp.mean(x * x, axis=-1, keepdims=True) + EPS)
    h = h * ln_ref[...]
    o_ref[...] = jnp.dot(h.astype(jnp.bfloat16), w_ref[...],
                         preferred_element_type=jnp.float32).astype(jnp.bfloat16)


BQ = 512
BK = 512
NC = S // BQ


def _flash_kernel(q_ref, k_ref, v_ref, wo_ref, x_ref, o_ref,
                  m_scr, l_scr, acc_scr, oall_scr, mask_scr):
    ci = pl.program_id(0)
    hp = pl.program_id(1)
    cj = pl.program_id(2)
    scale = 1.0 / np.sqrt(HD)
    B2 = 2 * BQ

    @pl.when((ci == 0) & (hp == 0) & (cj == 0))
    def _():
        rl = jax.lax.broadcasted_iota(jnp.int32, (B2, BK), 0) % BQ
        cl = jax.lax.broadcasted_iota(jnp.int32, (B2, BK), 1)
        mask_scr[...] = jnp.where(cl <= rl, 0.0, -1e30)

    @pl.when(cj == 0)
    def _():
        m_scr[...] = jnp.full((B2, 128), -1e30, jnp.float32)
        l_scr[...] = jnp.zeros((B2, 128), jnp.float32)
        acc_scr[...] = jnp.zeros((B2, HD), jnp.float32)

    @pl.when(cj <= ci)
    def _():
        q2 = jnp.concatenate([q_ref[0], q_ref[1]], axis=0)
        s = jax.lax.dot_general(q2, k_ref[0], (((1,), (1,)), ((), ())),
                                preferred_element_type=jnp.float32) * scale
        s = jax.lax.cond(cj == ci, lambda: s + mask_scr[...], lambda: s)
        m_old = m_scr[:, 0:1]
        m_new = jnp.maximum(m_old, jnp.max(s, axis=-1, keepdims=True))
        p = jnp.exp(s - m_new)
        corr = jnp.exp(m_old - m_new)
        l_new = l_scr[:, 0:1] * corr + jnp.sum(p, axis=-1, keepdims=True)
        m_scr[...] = jnp.broadcast_to(m_new, (B2, 128))
        l_scr[...] = jnp.broadcast_to(l_new, (B2, 128))
        pv = jax.lax.dot_general(p.astype(jnp.bfloat16), v_ref[0],
                                 (((1,), (0,)), ((), ())),
                                 preferred_element_type=jnp.float32)
        acc_scr[...] = acc_scr[...] * corr + pv

    @pl.when(cj == ci)
    def _():
        o_h = acc_scr[...] / l_scr[:, 0:1]
        o_pair = jnp.concatenate([o_h[:BQ], o_h[BQ:]], axis=1)
        oall_scr[:, pl.ds(hp * 2 * HD, 2 * HD)] = o_pair.astype(jnp.bfloat16)

    @pl.when((hp == NKV - 1) & (cj == ci))
    def _():
        o_ref[...] = x_ref[...] + jax.lax.dot_general(
            oall_scr[...], wo_ref[...], (((1,), (0,)), ((), ())),
            preferred_element_type=jnp.float32)


def _router_kernel(x_ref, ln_ref, gw_ref, h_ref, comb_ref, aux_ref):
    x = x_ref[...]
    h = x * jax.lax.rsqrt(jnp.mean(x * x, axis=-1, keepdims=True) + EPS)
    h = h * ln_ref[...]
    h_ref[...] = h.astype(jnp.bfloat16)
    logits = jax.lax.dot_general(h, gw_ref[...], (((1,), (0,)), ((), ())),
                                 precision=jax.lax.Precision.HIGHEST,
                                 preferred_element_type=jnp.float32)
    lane = jax.lax.broadcasted_iota(jnp.int32, (S, EP), 1)
    logits = jnp.where(lane < E, logits, -jnp.inf)
    lm = jnp.max(logits, axis=-1, keepdims=True)
    ex = jnp.exp(logits - lm)
    probs = ex / jnp.sum(ex, axis=-1, keepdims=True)
    m1 = jnp.max(probs, axis=-1, keepdims=True)
    i1 = jnp.min(jnp.where(probs == m1, lane, EP), axis=-1, keepdims=True)
    mask1 = lane == i1
    p2 = jnp.where(mask1, -jnp.inf, probs)
    m2 = jnp.max(p2, axis=-1, keepdims=True)
    i2 = jnp.min(jnp.where(p2 == m2, lane, EP), axis=-1, keepdims=True)
    mask2 = lane == i2
    comb = jnp.where(mask1, m1, 0.0) + jnp.where(mask2, m2, 0.0)
    comb_ref[...] = comb + jnp.where(lane == E, 1.0, 0.0)
    counts = jnp.sum(jnp.where(mask1 | mask2, 1.0, 0.0), axis=0, keepdims=True)
    importance = jnp.mean(probs, axis=0, keepdims=True)
    aux = jnp.sum(importance * counts) * (E / (S * 2.0))
    aux_ref[...] = jnp.full((1, 128), aux, jnp.float32)


def _moe_dense_kernel(h_ref, comb_ref, x_ref, w1_ref, w2_ref, o_ref):
    e = pl.program_id(0)
    c = pl.program_id(1)

    @pl.when((e == 0) & (c == 0))
    def _():
        o_ref[...] = x_ref[...]

    h1 = jax.lax.dot_general(h_ref[...], w1_ref[0], (((1,), (1,)), ((), ())),
                             preferred_element_type=jnp.float32)
    h1 = h1 * jax.nn.sigmoid(h1)
    lane = jax.lax.broadcasted_iota(jnp.int32, (S, EP), 1)
    cw = jnp.sum(jnp.where(lane == e, comb_ref[...], 0.0), axis=-1,
                 keepdims=True)
    h1 = (h1 * cw).astype(jnp.bfloat16)
    o_ref[...] += jax.lax.dot_general(h1, w2_ref[0], (((1,), (1,)), ((), ())),
                                      preferred_element_type=jnp.float32)


def kernel(x, wq, wk, wv, wo, gate_w, w1, w2, sw1, sw2, ln1, ln2):
    xf = x[0]
    wqkv_t = jnp.concatenate([wq, wk, wv], axis=0).T.astype(jnp.bfloat16)
    wo_t = wo.T.astype(jnp.bfloat16)
    gw_t = jnp.pad(gate_w, ((0, EP - E), (0, 0))).T
    w1s = jnp.concatenate([w1, sw1[None]], axis=0).astype(jnp.bfloat16)
    w2s = jnp.concatenate([w2, sw2[None]], axis=0).astype(jnp.bfloat16)

    qkv = pl.pallas_call(
        _rms_mm_kernel,
        out_shape=jax.ShapeDtypeStruct((S, NH * HD + 2 * NKV * HD),
                                       jnp.bfloat16),
    )(xf, ln1.reshape(1, H), wqkv_t)

    q3 = qkv[:, :NH * HD].reshape(S, NH, HD).transpose(1, 0, 2)
    k3 = qkv[:, NH * HD:(NH + NKV) * HD].reshape(S, NKV, HD).transpose(1, 0, 2)
    v3 = qkv[:, (NH + NKV) * HD:].reshape(S, NKV, HD).transpose(1, 0, 2)

    x2 = pl.pallas_call(
        _flash_kernel,
        grid=(NC, NKV, S // BK),
        in_specs=[
            pl.BlockSpec((2, BQ, HD), lambda ci, hp, cj: (hp, ci, 0)),
            pl.BlockSpec((1, BK, HD), lambda ci, hp, cj: (hp, cj, 0)),
            pl.BlockSpec((1, BK, HD), lambda ci, hp, cj: (hp, cj, 0)),
            pl.BlockSpec((NH * HD, H), lambda ci, hp, cj: (0, 0)),
            pl.BlockSpec((BQ, H), lambda ci, hp, cj: (ci, 0)),
        ],
        out_specs=pl.BlockSpec((BQ, H), lambda ci, hp, cj: (ci, 0)),
        out_shape=jax.ShapeDtypeStruct((S, H), jnp.float32),
        scratch_shapes=[
            pltpu.VMEM((2 * BQ, 128), jnp.float32),
            pltpu.VMEM((2 * BQ, 128), jnp.float32),
            pltpu.VMEM((2 * BQ, HD), jnp.float32),
            pltpu.VMEM((BQ, NH * HD), jnp.bfloat16),
            pltpu.VMEM((2 * BQ, BK), jnp.float32),
        ],
        compiler_params=pltpu.CompilerParams(
            dimension_semantics=("arbitrary", "arbitrary", "arbitrary")),
    )(q3, k3, v3, wo_t, xf)

    h2, comb, aux = pl.pallas_call(
        _router_kernel,
        out_shape=[
            jax.ShapeDtypeStruct((S, H), jnp.bfloat16),
            jax.ShapeDtypeStruct((S, EP), jnp.float32),
            jax.ShapeDtypeStruct((1, 128), jnp.float32),
        ],
    )(x2, ln2.reshape(1, H), gw_t)

    IC = I // 2
    y = pl.pallas_call(
        _moe_dense_kernel,
        grid=(E + 1, 2),
        in_specs=[
            pl.BlockSpec((S, H), lambda e, c: (0, 0)),
            pl.BlockSpec((S, EP), lambda e, c: (0, 0)),
            pl.BlockSpec((S, H), lambda e, c: (0, 0)),
            pl.BlockSpec((1, IC, H), lambda e, c: (e, c, 0)),
            pl.BlockSpec((1, H, IC), lambda e, c: (e, 0, c)),
        ],
        out_specs=pl.BlockSpec((S, H), lambda e, c: (0, 0)),
        out_shape=jax.ShapeDtypeStruct((S, H), jnp.float32),
        compiler_params=pltpu.CompilerParams(
            dimension_semantics=("arbitrary", "arbitrary")),
    )(h2, comb, x2, w1s, w2s)

    return y.reshape(1, S, H), aux[0, 0]
